# baseline (speedup 1.0000x reference)
import functools

import jax
import jax.numpy as jnp
from jax import lax
from jax.experimental import pallas as pl
from jax.experimental.pallas import tpu as pltpu
from jax.experimental.pallas import tpu_sc as plsc

_N_EMB = 8192
_D = 32
_N_TOK = 16384

_TT = 128
_GRID = _N_TOK // _TT

_NC = 2
_NS = 16
_NW = _NC * _NS
_BPW = _N_TOK // _NW
_CHUNK = 128
_NCHUNK = _BPW // _CHUNK


def _argmin_body(ze_ref, embt_ref, ix_ref, loss_ref):
    ze = ze_ref[...]
    embt = embt_ref[...]
    a2 = jnp.sum(ze * ze, axis=1, keepdims=True)
    b2 = jnp.sum(embt * embt, axis=0, keepdims=True)
    dot = lax.dot_general(
        ze.astype(jnp.bfloat16), embt.astype(jnp.bfloat16),
        (((1,), (0,)), ((), ())),
        preferred_element_type=jnp.float32,
    )
    d2 = a2 + b2 - 2.0 * dot
    dist = jnp.sqrt(jnp.maximum(d2, 0.0))

    half = _N_EMB // 2
    d0 = dist[:, :half]
    d1 = dist[:, half:]
    m0 = jnp.min(d0, axis=1, keepdims=True)
    m1 = jnp.min(d1, axis=1, keepdims=True)
    iota = lax.broadcasted_iota(jnp.int32, d0.shape, 1)
    i0 = jnp.min(jnp.where(d0 <= m0, iota, half), axis=1).reshape(_TT, 1)
    i1 = jnp.min(jnp.where(d1 <= m1, iota, half), axis=1).reshape(_TT, 1)
    q0 = m0.astype(jnp.bfloat16).astype(jnp.float32)
    use1 = m1 < q0
    ix = jnp.where(use1, i1 + half, i0)
    ix_ref[...] = ix

    mch = jnp.where(use1, m1, m0)

    @pl.when(pl.program_id(0) == 0)
    def _():
        loss_ref[0, 0] = 0.0

    loss_ref[0, 0] += jnp.sum(mch * mch)


_argmin_call = pl.pallas_call(
    _argmin_body,
    grid=(_GRID,),
    in_specs=[
        pl.BlockSpec((_TT, _D), lambda i: (i, 0)),
        pl.BlockSpec((_D, _N_EMB), lambda i: (0, 0)),
    ],
    out_specs=[
        pl.BlockSpec((_TT, 1), lambda i: (i, 0)),
        pl.BlockSpec(memory_space=pltpu.SMEM, block_shape=(1, 1),
                     index_map=lambda i: (0, 0)),
    ],
    out_shape=[
        jax.ShapeDtypeStruct((_N_TOK, 1), jnp.int32),
        jax.ShapeDtypeStruct((1, 1), jnp.float32),
    ],
)


_DPAD = 128


@functools.cache
def _make_sc_gather():
    mesh = plsc.VectorSubcoreMesh(core_axis_name="c", subcore_axis_name="s")

    @functools.partial(
        pl.kernel,
        mesh=mesh,
        out_type=jax.ShapeDtypeStruct((_N_TOK, _DPAD), jnp.float32),
        scratch_types=[
            pltpu.VMEM((_NCHUNK, _CHUNK), jnp.int32),
            pltpu.VMEM((_BPW, _DPAD), jnp.float32),
            pltpu.SemaphoreType.DMA,
        ],
    )
    def gather(emb_hbm, idx_hbm, out_hbm, idx_v, rows_v, sem):
        wid = lax.axis_index("s") * _NC + lax.axis_index("c")
        pltpu.sync_copy(idx_hbm.at[wid], idx_v)
        copies = [
            pltpu.async_copy(emb_hbm.at[idx_v.at[j]],
                             rows_v.at[pl.ds(j * _CHUNK, _CHUNK)], sem)
            for j in range(_NCHUNK)
        ]
        for c in copies:
            c.wait()
        pltpu.sync_copy(rows_v, out_hbm.at[pl.ds(wid * _BPW, _BPW)])

    return gather


def kernel(ze, emb):
    ix2, loss = _argmin_call(ze, emb.T)
    emb_ix = ix2.reshape(_N_TOK)
    idx3 = emb_ix.reshape(_NW, _NCHUNK, _CHUNK)
    emb_p = jnp.pad(emb, ((0, 0), (0, _DPAD - _D)))
    zq = _make_sc_gather()(emb_p, idx3)[:, :_D]
    vq_loss = loss[0, 0] * (2.0 / (_N_TOK * _D))
    return (zq, emb_ix, vq_loss)


# speedup vs baseline: 1.0792x; 1.0792x over previous
import functools

import jax
import jax.numpy as jnp
from jax import lax
from jax.experimental import pallas as pl
from jax.experimental.pallas import tpu as pltpu
from jax.experimental.pallas import tpu_sc as plsc

_N_EMB = 8192
_D = 32
_N_TOK = 16384

_TT = 256
_GRID = _N_TOK // _TT

_NC = 2
_NS = 16
_NW = _NC * _NS
_BPW = _N_TOK // _NW
_CHUNK = 128
_NCHUNK = _BPW // _CHUNK


def _argmin_body(ze_ref, embt_ref, ix_ref, loss_ref):
    ze = ze_ref[...]
    embt = embt_ref[...]
    a2 = jnp.sum(ze * ze, axis=1, keepdims=True)
    b2 = jnp.sum(embt * embt, axis=0, keepdims=True)
    dot = lax.dot_general(
        ze.astype(jnp.bfloat16), embt.astype(jnp.bfloat16),
        (((1,), (0,)), ((), ())),
        preferred_element_type=jnp.float32,
    )
    d2 = a2 + b2 - 2.0 * dot
    dist = jnp.sqrt(jnp.maximum(d2, 0.0))

    half = _N_EMB // 2
    d0 = dist[:, :half]
    d1 = dist[:, half:]
    m0 = jnp.min(d0, axis=1, keepdims=True)
    m1 = jnp.min(d1, axis=1, keepdims=True)
    iota = lax.broadcasted_iota(jnp.int32, d0.shape, 1)
    i0 = jnp.min(jnp.where(d0 <= m0, iota, half), axis=1).reshape(_TT, 1)
    i1 = jnp.min(jnp.where(d1 <= m1, iota, half), axis=1).reshape(_TT, 1)
    q0 = m0.astype(jnp.bfloat16).astype(jnp.float32)
    use1 = m1 < q0
    ix = jnp.where(use1, i1 + half, i0)
    ix_ref[...] = ix

    mch = jnp.where(use1, m1, m0)

    @pl.when(pl.program_id(0) == 0)
    def _():
        loss_ref[0, 0] = 0.0

    loss_ref[0, 0] += jnp.sum(mch * mch)


_argmin_call = pl.pallas_call(
    _argmin_body,
    grid=(_GRID,),
    in_specs=[
        pl.BlockSpec((_TT, _D), lambda i: (i, 0)),
        pl.BlockSpec((_D, _N_EMB), lambda i: (0, 0)),
    ],
    out_specs=[
        pl.BlockSpec((_TT, 1), lambda i: (i, 0)),
        pl.BlockSpec(memory_space=pltpu.SMEM, block_shape=(1, 1),
                     index_map=lambda i: (0, 0)),
    ],
    out_shape=[
        jax.ShapeDtypeStruct((_N_TOK, 1), jnp.int32),
        jax.ShapeDtypeStruct((1, 1), jnp.float32),
    ],
)


_DPAD = 128


@functools.cache
def _make_sc_gather():
    mesh = plsc.VectorSubcoreMesh(core_axis_name="c", subcore_axis_name="s")

    @functools.partial(
        pl.kernel,
        mesh=mesh,
        out_type=jax.ShapeDtypeStruct((_N_TOK, _DPAD), jnp.float32),
        scratch_types=[
            pltpu.VMEM((_NCHUNK, _CHUNK), jnp.int32),
            pltpu.VMEM((_BPW, _DPAD), jnp.float32),
            pltpu.SemaphoreType.DMA,
        ],
    )
    def gather(emb_hbm, idx_hbm, out_hbm, idx_v, rows_v, sem):
        wid = lax.axis_index("s") * _NC + lax.axis_index("c")
        pltpu.sync_copy(idx_hbm.at[wid], idx_v)
        copies = [
            pltpu.async_copy(emb_hbm.at[idx_v.at[j]],
                             rows_v.at[pl.ds(j * _CHUNK, _CHUNK)], sem)
            for j in range(_NCHUNK)
        ]
        for c in copies:
            c.wait()
        pltpu.sync_copy(rows_v, out_hbm.at[pl.ds(wid * _BPW, _BPW)])

    return gather


def kernel(ze, emb):
    ix2, loss = _argmin_call(ze, emb.T)
    emb_ix = ix2.reshape(_N_TOK)
    idx3 = emb_ix.reshape(_NW, _NCHUNK, _CHUNK)
    emb_p = jnp.pad(emb, ((0, 0), (0, _DPAD - _D)))
    zq = _make_sc_gather()(emb_p, idx3)[:, :_D]
    vq_loss = loss[0, 0] * (2.0 / (_N_TOK * _D))
    return (zq, emb_ix, vq_loss)


# speedup vs baseline: 1.1274x; 1.0446x over previous
import functools

import jax
import jax.numpy as jnp
from jax import lax
from jax.experimental import pallas as pl
from jax.experimental.pallas import tpu as pltpu
from jax.experimental.pallas import tpu_sc as plsc

_N_EMB = 8192
_D = 32
_N_TOK = 16384

_TT = 512
_GRID = _N_TOK // _TT

_NC = 2
_NS = 16
_NW = _NC * _NS
_BPW = _N_TOK // _NW
_CHUNK = 128
_NCHUNK = _BPW // _CHUNK


def _argmin_body(ze_ref, embt_ref, ix_ref, loss_ref):
    ze = ze_ref[...]
    embt = embt_ref[...]
    a2 = jnp.sum(ze * ze, axis=1, keepdims=True)
    b2 = jnp.sum(embt * embt, axis=0, keepdims=True)
    dot = lax.dot_general(
        ze.astype(jnp.bfloat16), embt.astype(jnp.bfloat16),
        (((1,), (0,)), ((), ())),
        preferred_element_type=jnp.float32,
    )
    d2 = a2 + b2 - 2.0 * dot
    dist = jnp.sqrt(jnp.maximum(d2, 0.0))

    half = _N_EMB // 2
    d0 = dist[:, :half]
    d1 = dist[:, half:]
    m0 = jnp.min(d0, axis=1, keepdims=True)
    m1 = jnp.min(d1, axis=1, keepdims=True)
    iota = lax.broadcasted_iota(jnp.int32, d0.shape, 1)
    i0 = jnp.min(jnp.where(d0 <= m0, iota, half), axis=1).reshape(_TT, 1)
    i1 = jnp.min(jnp.where(d1 <= m1, iota, half), axis=1).reshape(_TT, 1)
    q0 = m0.astype(jnp.bfloat16).astype(jnp.float32)
    use1 = m1 < q0
    ix = jnp.where(use1, i1 + half, i0)
    ix_ref[...] = ix

    mch = jnp.where(use1, m1, m0)

    @pl.when(pl.program_id(0) == 0)
    def _():
        loss_ref[0, 0] = 0.0

    loss_ref[0, 0] += jnp.sum(mch * mch)


_argmin_call = pl.pallas_call(
    _argmin_body,
    grid=(_GRID,),
    in_specs=[
        pl.BlockSpec((_TT, _D), lambda i: (i, 0)),
        pl.BlockSpec((_D, _N_EMB), lambda i: (0, 0)),
    ],
    out_specs=[
        pl.BlockSpec((_TT, 1), lambda i: (i, 0)),
        pl.BlockSpec(memory_space=pltpu.SMEM, block_shape=(1, 1),
                     index_map=lambda i: (0, 0)),
    ],
    out_shape=[
        jax.ShapeDtypeStruct((_N_TOK, 1), jnp.int32),
        jax.ShapeDtypeStruct((1, 1), jnp.float32),
    ],
)


_DPAD = 128


@functools.cache
def _make_sc_gather():
    mesh = plsc.VectorSubcoreMesh(core_axis_name="c", subcore_axis_name="s")

    @functools.partial(
        pl.kernel,
        mesh=mesh,
        out_type=jax.ShapeDtypeStruct((_N_TOK, _DPAD), jnp.float32),
        scratch_types=[
            pltpu.VMEM((_NCHUNK, _CHUNK), jnp.int32),
            pltpu.VMEM((_BPW, _DPAD), jnp.float32),
            pltpu.SemaphoreType.DMA,
        ],
    )
    def gather(emb_hbm, idx_hbm, out_hbm, idx_v, rows_v, sem):
        wid = lax.axis_index("s") * _NC + lax.axis_index("c")
        pltpu.sync_copy(idx_hbm.at[wid], idx_v)
        copies = [
            pltpu.async_copy(emb_hbm.at[idx_v.at[j]],
                             rows_v.at[pl.ds(j * _CHUNK, _CHUNK)], sem)
            for j in range(_NCHUNK)
        ]
        for c in copies:
            c.wait()
        pltpu.sync_copy(rows_v, out_hbm.at[pl.ds(wid * _BPW, _BPW)])

    return gather


def kernel(ze, emb):
    ix2, loss = _argmin_call(ze, emb.T)
    emb_ix = ix2.reshape(_N_TOK)
    idx3 = emb_ix.reshape(_NW, _NCHUNK, _CHUNK)
    emb_p = jnp.pad(emb, ((0, 0), (0, _DPAD - _D)))
    zq = _make_sc_gather()(emb_p, idx3)[:, :_D]
    vq_loss = loss[0, 0] * (2.0 / (_N_TOK * _D))
    return (zq, emb_ix, vq_loss)


# speedup vs baseline: 1.1926x; 1.0579x over previous
import functools

import jax
import jax.numpy as jnp
from jax import lax
from jax.experimental import pallas as pl
from jax.experimental.pallas import tpu as pltpu
from jax.experimental.pallas import tpu_sc as plsc

_N_EMB = 8192
_D = 32
_N_TOK = 16384

_TT = 1024
_GRID = _N_TOK // _TT

_NC = 2
_NS = 16
_NW = _NC * _NS
_BPW = _N_TOK // _NW
_CHUNK = 128
_NCHUNK = _BPW // _CHUNK


def _argmin_body(ze_ref, embt_ref, ix_ref, loss_ref):
    ze = ze_ref[...]
    embt = embt_ref[...]
    a2 = jnp.sum(ze * ze, axis=1, keepdims=True)
    b2 = jnp.sum(embt * embt, axis=0, keepdims=True)
    dot = lax.dot_general(
        ze.astype(jnp.bfloat16), embt.astype(jnp.bfloat16),
        (((1,), (0,)), ((), ())),
        preferred_element_type=jnp.float32,
    )
    d2 = a2 + b2 - 2.0 * dot
    dist = jnp.sqrt(jnp.maximum(d2, 0.0))

    half = _N_EMB // 2
    d0 = dist[:, :half]
    d1 = dist[:, half:]
    m0 = jnp.min(d0, axis=1, keepdims=True)
    m1 = jnp.min(d1, axis=1, keepdims=True)
    iota = lax.broadcasted_iota(jnp.int32, d0.shape, 1)
    i0 = jnp.min(jnp.where(d0 <= m0, iota, half), axis=1).reshape(_TT, 1)
    i1 = jnp.min(jnp.where(d1 <= m1, iota, half), axis=1).reshape(_TT, 1)
    q0 = m0.astype(jnp.bfloat16).astype(jnp.float32)
    use1 = m1 < q0
    ix = jnp.where(use1, i1 + half, i0)
    ix_ref[...] = ix

    mch = jnp.where(use1, m1, m0)

    @pl.when(pl.program_id(0) == 0)
    def _():
        loss_ref[0, 0] = 0.0

    loss_ref[0, 0] += jnp.sum(mch * mch)


_argmin_call = pl.pallas_call(
    _argmin_body,
    grid=(_GRID,),
    in_specs=[
        pl.BlockSpec((_TT, _D), lambda i: (i, 0)),
        pl.BlockSpec((_D, _N_EMB), lambda i: (0, 0)),
    ],
    out_specs=[
        pl.BlockSpec((_TT, 1), lambda i: (i, 0)),
        pl.BlockSpec(memory_space=pltpu.SMEM, block_shape=(1, 1),
                     index_map=lambda i: (0, 0)),
    ],
    out_shape=[
        jax.ShapeDtypeStruct((_N_TOK, 1), jnp.int32),
        jax.ShapeDtypeStruct((1, 1), jnp.float32),
    ],
)


_DPAD = 128


@functools.cache
def _make_sc_gather():
    mesh = plsc.VectorSubcoreMesh(core_axis_name="c", subcore_axis_name="s")

    @functools.partial(
        pl.kernel,
        mesh=mesh,
        out_type=jax.ShapeDtypeStruct((_N_TOK, _DPAD), jnp.float32),
        scratch_types=[
            pltpu.VMEM((_NCHUNK, _CHUNK), jnp.int32),
            pltpu.VMEM((_BPW, _DPAD), jnp.float32),
            pltpu.SemaphoreType.DMA,
        ],
    )
    def gather(emb_hbm, idx_hbm, out_hbm, idx_v, rows_v, sem):
        wid = lax.axis_index("s") * _NC + lax.axis_index("c")
        pltpu.sync_copy(idx_hbm.at[wid], idx_v)
        copies = [
            pltpu.async_copy(emb_hbm.at[idx_v.at[j]],
                             rows_v.at[pl.ds(j * _CHUNK, _CHUNK)], sem)
            for j in range(_NCHUNK)
        ]
        for c in copies:
            c.wait()
        pltpu.sync_copy(rows_v, out_hbm.at[pl.ds(wid * _BPW, _BPW)])

    return gather


def kernel(ze, emb):
    ix2, loss = _argmin_call(ze, emb.T)
    emb_ix = ix2.reshape(_N_TOK)
    idx3 = emb_ix.reshape(_NW, _NCHUNK, _CHUNK)
    emb_p = jnp.pad(emb, ((0, 0), (0, _DPAD - _D)))
    zq = _make_sc_gather()(emb_p, idx3)[:, :_D]
    vq_loss = loss[0, 0] * (2.0 / (_N_TOK * _D))
    return (zq, emb_ix, vq_loss)
